# baseline (speedup 1.0000x reference)
import functools

import jax
import jax.numpy as jnp
from jax import lax
from jax.experimental import pallas as pl
from jax.experimental.pallas import tpu as pltpu
from jax.experimental.pallas import tpu_sc as plsc

B = 64
L = 448
D = 1024
P = 7
N = B * L

NC, NS = 2, 16
NW = NC * NS
BPW = N // NW
CHUNK = 64
NCHUNK = BPW // CHUNK

_MESH = plsc.VectorSubcoreMesh(
    core_axis_name="c", subcore_axis_name="s", num_cores=NC, num_subcores=NS
)


@functools.partial(
    pl.kernel,
    out_type=jax.ShapeDtypeStruct((N, D), jnp.float32),
    mesh=_MESH,
    scratch_types=[
        pltpu.VMEM((BPW,), jnp.int32),
        pltpu.VMEM((CHUNK, D), jnp.float32),
        pltpu.SemaphoreType.DMA,
    ],
)
def _sc_gather(table_hbm, idx_hbm, out_hbm, idx_v, rows_v, sem):
    wid = lax.axis_index("s") * NC + lax.axis_index("c")
    base = wid * BPW
    pltpu.sync_copy(idx_hbm.at[pl.ds(base, BPW)], idx_v)
    for c in range(NCHUNK):
        pltpu.async_copy(
            table_hbm.at[idx_v.at[pl.ds(c * CHUNK, CHUNK)]], rows_v, sem
        ).wait()
        pltpu.sync_copy(rows_v, out_hbm.at[pl.ds(base + c * CHUNK, CHUNK)])


def _tc_fuse_body(tok_ref, pos_ref, pros_ref, w_ref, b_ref, out_ref):
    proj = jax.lax.dot_general(
        pros_ref[...], w_ref[...],
        dimension_numbers=(((1,), (0,)), ((), ())),
        preferred_element_type=jnp.float32,
    )
    out_ref[...] = tok_ref[...] + pos_ref[...] + proj + b_ref[...]


def kernel(token_ids, prosody_features, token_table, pos_table, proj_w, proj_b):
    ids = token_ids.reshape(N).astype(jnp.int32)
    tok_emb = _sc_gather(token_table, ids)
    pros = prosody_features.reshape(N, P)
    out = pl.pallas_call(
        _tc_fuse_body,
        grid=(B,),
        in_specs=[
            pl.BlockSpec((L, D), lambda b: (b, 0)),
            pl.BlockSpec((L, D), lambda b: (0, 0)),
            pl.BlockSpec((L, P), lambda b: (b, 0)),
            pl.BlockSpec((P, D), lambda b: (0, 0)),
            pl.BlockSpec((1, D), lambda b: (0, 0)),
        ],
        out_specs=pl.BlockSpec((L, D), lambda b: (b, 0)),
        out_shape=jax.ShapeDtypeStruct((N, D), jnp.float32),
    )(tok_emb, pos_table, pros, proj_w, proj_b.reshape(1, D))
    return out.reshape(B, L, D)
